# baseline (speedup 1.0000x reference)
import math

import jax
import jax.numpy as jnp
from jax.experimental import pallas as pl
from jax.experimental.pallas import tpu as pltpu

U = 64
B = 128
SIG = 64
RULE = 64
CELLS = U * B
SUPPORT_MIN = 1e-4
PRIOR_MIN_POP = 4.0
PRIOR_SOFT_CAP = 0.75
USAGE_SCALE = 0.5
CONF_SCALE = 0.5
SIG_SCALE = 1.0
RET_TEMP = 1.0
SPARSE_BOOST = 1.0

T = 128


def _prologue(sigT_ref, sup_ref, conf_ref, sn_out, usp_out, ccp_out, vf_out, cst_out):
    f32 = jnp.float32
    sup = sup_ref[:]
    valid = sup > SUPPORT_MIN
    vf = valid.astype(f32)
    occ = jnp.sum(vf, keepdims=True)
    ps = jnp.clip(occ / PRIOR_MIN_POP, 0.0, 1.0)
    sp = jnp.log1p(sup)
    sp = sp / jnp.maximum(jnp.max(sp, keepdims=True), 1.0)
    sp = jnp.clip(sp * ps, 0.0, PRIOR_SOFT_CAP)
    cp = conf_ref[:] / jnp.maximum(jnp.max(conf_ref[:], keepdims=True), 1e-6)
    cp = jnp.clip(cp * ps, 0.0, PRIOR_SOFT_CAP)
    eff_temp = RET_TEMP * (1.0 + SPARSE_BOOST * (1.0 - ps))
    it = 1.0 / jnp.maximum(eff_temp, 1e-6)

    se = sigT_ref[:] + 1e-6
    nrm = jnp.maximum(jnp.sqrt(jnp.sum(se * se, axis=0, keepdims=True)), 1e-12)
    sn_out[:] = se / nrm
    usp_out[:] = USAGE_SCALE * sp
    ccp_out[:] = CONF_SCALE * cp
    vf_out[:] = vf
    lane = jax.lax.broadcasted_iota(jnp.int32, (1, 128), 1)
    cst_out[:] = ps * (lane == 0) + it * (lane == 1)


def _main(qu_ref, qb_ref, qs_ref, sn_ref, eu_ref, usp_ref, ccp_ref, vf_ref,
          cst_ref, drsp_ref, dr_out, sig_out, conf_out, w_out, tw_out):
    f32 = jnp.float32
    ps = cst_ref[0:1, 0:1]
    it = cst_ref[0:1, 1:2]

    qs = qs_ref[:]
    qn = jnp.maximum(jnp.sqrt(jnp.sum(qs * qs, axis=1, keepdims=True)), 1e-12)
    qsn = qs / qn
    sig_score = 0.5 * (1.0 + jnp.dot(qsn, sn_ref[:],
                                     preferred_element_type=f32))

    qu_f = jnp.dot(qu_ref[:], eu_ref[:], preferred_element_type=f32,
                   precision=jax.lax.Precision.HIGHEST)
    qb_f = jnp.concatenate([qb_ref[:]] * U, axis=1)
    jl = jnp.log(jnp.maximum(qu_f * qb_f, 1e-6))
    logits = jl + usp_ref[:] + ccp_ref[:] + SIG_SCALE * sig_score
    vf = vf_ref[:]
    z = jnp.where(vf > 0, logits, -1e9) * it

    m = jnp.max(z, axis=1, keepdims=True)
    e = jnp.exp(z - m)
    w_pre = e / jnp.sum(e, axis=1, keepdims=True)
    wv = w_pre * vf
    w = wv / jnp.maximum(jnp.sum(wv, axis=1, keepdims=True), 1e-6)

    out = jnp.dot(w, drsp_ref[:], preferred_element_type=f32)
    dr_out[:] = out[:, :RULE]
    sig_out[:] = out[:, RULE:]

    tw = jnp.max(w, axis=1, keepdims=True)
    iota = jax.lax.broadcasted_iota(jnp.int32, (1, CELLS), 1)
    ti = jnp.min(jnp.where(w == tw, iota, CELLS), axis=1, keepdims=True)
    oh = iota == ti
    top_c = 2.0 * jnp.sum(jnp.where(oh, ccp_ref[:], 0.0), axis=1, keepdims=True)
    top_s = jnp.sum(jnp.where(oh, sig_score, 0.0), axis=1, keepdims=True)
    tw_out[:] = tw
    conf_out[:] = jnp.clip(tw * top_c * top_s * ps, 0.0, 1.0)

    w_out[...] = w.reshape(1, T, U, B)


@jax.jit
def kernel(q_u, q_b, q_sigma, delta_rule_proto, signature_proto, support_ema, ema_conf):
    f32 = jnp.float32
    lead = q_u.shape[:-1]
    n = 1
    for d in lead:
        n *= d
    qu2 = q_u.reshape(n, U)
    qb2 = q_b.reshape(n, B)
    qs2 = q_sigma.reshape(n, SIG)
    sigT = signature_proto.reshape(CELLS, SIG).T
    drsp = jnp.concatenate(
        [delta_rule_proto.reshape(CELLS, RULE),
         signature_proto.reshape(CELLS, SIG)], axis=1)
    supF = support_ema.reshape(1, CELLS)
    confF = ema_conf.reshape(1, CELLS)
    eu = (jax.lax.broadcasted_iota(jnp.int32, (U, CELLS), 1) // B
          == jax.lax.broadcasted_iota(jnp.int32, (U, CELLS), 0)).astype(f32)

    sn, usp, ccp, vf, cst = pl.pallas_call(
        _prologue,
        in_specs=[pl.BlockSpec((SIG, CELLS), lambda: (0, 0)),
                  pl.BlockSpec((1, CELLS), lambda: (0, 0)),
                  pl.BlockSpec((1, CELLS), lambda: (0, 0))],
        out_specs=[pl.BlockSpec((SIG, CELLS), lambda: (0, 0)),
                   pl.BlockSpec((1, CELLS), lambda: (0, 0)),
                   pl.BlockSpec((1, CELLS), lambda: (0, 0)),
                   pl.BlockSpec((1, CELLS), lambda: (0, 0)),
                   pl.BlockSpec((1, 128), lambda: (0, 0))],
        out_shape=[
            jax.ShapeDtypeStruct((SIG, CELLS), f32),
            jax.ShapeDtypeStruct((1, CELLS), f32),
            jax.ShapeDtypeStruct((1, CELLS), f32),
            jax.ShapeDtypeStruct((1, CELLS), f32),
            jax.ShapeDtypeStruct((1, 128), f32),
        ],
    )(sigT, supF, confF)

    grid = (n // T,)
    steps_per_lead = lead[-1] // T
    full = lambda shape: pl.BlockSpec(shape, lambda i: tuple(0 for _ in shape))
    tok = lambda width: pl.BlockSpec((T, width), lambda i: (i, 0))

    dr, sig, mconf, w4, tw = pl.pallas_call(
        _main,
        grid=grid,
        in_specs=[
            tok(U), tok(B), tok(SIG),
            full((SIG, CELLS)), full((U, CELLS)),
            full((1, CELLS)), full((1, CELLS)), full((1, CELLS)),
            full((1, 128)), full((CELLS, 128)),
        ],
        out_specs=[
            tok(RULE), tok(SIG), tok(1),
            pl.BlockSpec((1, T, U, B),
                         lambda i: (i // steps_per_lead, i % steps_per_lead, 0, 0)),
            tok(1),
        ],
        out_shape=[
            jax.ShapeDtypeStruct((n, RULE), f32),
            jax.ShapeDtypeStruct((n, SIG), f32),
            jax.ShapeDtypeStruct((n, 1), f32),
            jax.ShapeDtypeStruct(lead + (U, B), f32),
            jax.ShapeDtypeStruct((n, 1), f32),
        ],
        compiler_params=pltpu.CompilerParams(
            dimension_semantics=("parallel",)),
    )(qu2, qb2, qs2, sn, eu, usp, ccp, vf, cst, drsp)

    return (
        dr.reshape(lead + (RULE,)),
        sig.reshape(lead + (SIG,)),
        mconf.reshape(lead + (1,)),
        w4,
        tw.reshape(lead + (1,)),
    )


# speedup vs baseline: 1.2243x; 1.2243x over previous
import math

import jax
import jax.numpy as jnp
from jax.experimental import pallas as pl
from jax.experimental.pallas import tpu as pltpu

U = 64
B = 128
SIG = 64
RULE = 64
CELLS = U * B
SUPPORT_MIN = 1e-4
PRIOR_MIN_POP = 4.0
PRIOR_SOFT_CAP = 0.75
USAGE_SCALE = 0.5
CONF_SCALE = 0.5
SIG_SCALE = 1.0
RET_TEMP = 1.0
SPARSE_BOOST = 1.0

T = 128


def _prologue(sigT_ref, sup_ref, conf_ref, sn_out, usp_out, ccp_out, vf_out, cst_out):
    f32 = jnp.float32
    sup = sup_ref[:]
    valid = sup > SUPPORT_MIN
    vf = valid.astype(f32)
    occ = jnp.sum(vf, keepdims=True)
    ps = jnp.clip(occ / PRIOR_MIN_POP, 0.0, 1.0)
    sp = jnp.log1p(sup)
    sp = sp / jnp.maximum(jnp.max(sp, keepdims=True), 1.0)
    sp = jnp.clip(sp * ps, 0.0, PRIOR_SOFT_CAP)
    cp = conf_ref[:] / jnp.maximum(jnp.max(conf_ref[:], keepdims=True), 1e-6)
    cp = jnp.clip(cp * ps, 0.0, PRIOR_SOFT_CAP)
    eff_temp = RET_TEMP * (1.0 + SPARSE_BOOST * (1.0 - ps))
    it = 1.0 / jnp.maximum(eff_temp, 1e-6)

    se = sigT_ref[:] + 1e-6
    nrm = jnp.maximum(jnp.sqrt(jnp.sum(se * se, axis=0, keepdims=True)), 1e-12)
    sn_out[:] = se / nrm
    usp_out[:] = USAGE_SCALE * sp
    ccp_out[:] = CONF_SCALE * cp
    vf_out[:] = vf
    lane = jax.lax.broadcasted_iota(jnp.int32, (1, 128), 1)
    cst_out[:] = ps * (lane == 0) + it * (lane == 1)


def _main(qu_ref, qb_ref, qs_ref, sn_ref, usp_ref, ccp_ref, vf_ref,
          cst_ref, drsp_ref, dr_out, sig_out, conf_out, w_out, tw_out):
    f32 = jnp.float32
    ps = cst_ref[0:1, 0:1]
    it = cst_ref[0:1, 1:2]

    qs = qs_ref[:]
    qn = jnp.maximum(jnp.sqrt(jnp.sum(qs * qs, axis=1, keepdims=True)), 1e-12)
    qsn = qs / qn
    sig_score = 0.5 * (1.0 + jnp.dot(qsn, sn_ref[:],
                                     preferred_element_type=f32))

    qu = qu_ref[:]
    qb = qb_ref[:]
    joint = jnp.concatenate([qu[:, u:u + 1] * qb for u in range(U)], axis=1)
    jl = jnp.log(jnp.maximum(joint, 1e-6))
    logits = jl + usp_ref[:] + ccp_ref[:] + SIG_SCALE * sig_score
    vf = vf_ref[:]
    z = jnp.where(vf > 0, logits, -1e9) * it

    m = jnp.max(z, axis=1, keepdims=True)
    e = jnp.exp(z - m)
    w_pre = e / jnp.sum(e, axis=1, keepdims=True)
    wv = w_pre * vf
    w = wv / jnp.maximum(jnp.sum(wv, axis=1, keepdims=True), 1e-6)

    out = jnp.dot(w, drsp_ref[:], preferred_element_type=f32)
    dr_out[:] = out[:, :RULE]
    sig_out[:] = out[:, RULE:]

    tw = jnp.max(w, axis=1, keepdims=True)
    iota = jax.lax.broadcasted_iota(jnp.int32, (1, CELLS), 1)
    ti = jnp.min(jnp.where(w == tw, iota, CELLS), axis=1, keepdims=True)
    oh = iota == ti
    top_c = 2.0 * jnp.sum(jnp.where(oh, ccp_ref[:], 0.0), axis=1, keepdims=True)
    top_s = jnp.sum(jnp.where(oh, sig_score, 0.0), axis=1, keepdims=True)
    tw_out[:] = tw
    conf_out[:] = jnp.clip(tw * top_c * top_s * ps, 0.0, 1.0)

    w_out[...] = w.reshape(1, T, U, B)


@jax.jit
def kernel(q_u, q_b, q_sigma, delta_rule_proto, signature_proto, support_ema, ema_conf):
    f32 = jnp.float32
    lead = q_u.shape[:-1]
    n = 1
    for d in lead:
        n *= d
    qu2 = q_u.reshape(n, U)
    qb2 = q_b.reshape(n, B)
    qs2 = q_sigma.reshape(n, SIG)
    sigT = signature_proto.reshape(CELLS, SIG).T
    drsp = jnp.concatenate(
        [delta_rule_proto.reshape(CELLS, RULE),
         signature_proto.reshape(CELLS, SIG)], axis=1)
    supF = support_ema.reshape(1, CELLS)
    confF = ema_conf.reshape(1, CELLS)
    sn, usp, ccp, vf, cst = pl.pallas_call(
        _prologue,
        in_specs=[pl.BlockSpec((SIG, CELLS), lambda: (0, 0)),
                  pl.BlockSpec((1, CELLS), lambda: (0, 0)),
                  pl.BlockSpec((1, CELLS), lambda: (0, 0))],
        out_specs=[pl.BlockSpec((SIG, CELLS), lambda: (0, 0)),
                   pl.BlockSpec((1, CELLS), lambda: (0, 0)),
                   pl.BlockSpec((1, CELLS), lambda: (0, 0)),
                   pl.BlockSpec((1, CELLS), lambda: (0, 0)),
                   pl.BlockSpec((1, 128), lambda: (0, 0))],
        out_shape=[
            jax.ShapeDtypeStruct((SIG, CELLS), f32),
            jax.ShapeDtypeStruct((1, CELLS), f32),
            jax.ShapeDtypeStruct((1, CELLS), f32),
            jax.ShapeDtypeStruct((1, CELLS), f32),
            jax.ShapeDtypeStruct((1, 128), f32),
        ],
    )(sigT, supF, confF)

    grid = (n // T,)
    steps_per_lead = lead[-1] // T
    full = lambda shape: pl.BlockSpec(shape, lambda i: tuple(0 for _ in shape))
    tok = lambda width: pl.BlockSpec((T, width), lambda i: (i, 0))

    dr, sig, mconf, w4, tw = pl.pallas_call(
        _main,
        grid=grid,
        in_specs=[
            tok(U), tok(B), tok(SIG),
            full((SIG, CELLS)),
            full((1, CELLS)), full((1, CELLS)), full((1, CELLS)),
            full((1, 128)), full((CELLS, 128)),
        ],
        out_specs=[
            tok(RULE), tok(SIG), tok(1),
            pl.BlockSpec((1, T, U, B),
                         lambda i: (i // steps_per_lead, i % steps_per_lead, 0, 0)),
            tok(1),
        ],
        out_shape=[
            jax.ShapeDtypeStruct((n, RULE), f32),
            jax.ShapeDtypeStruct((n, SIG), f32),
            jax.ShapeDtypeStruct((n, 1), f32),
            jax.ShapeDtypeStruct(lead + (U, B), f32),
            jax.ShapeDtypeStruct((n, 1), f32),
        ],
        compiler_params=pltpu.CompilerParams(
            dimension_semantics=("parallel",)),
    )(qu2, qb2, qs2, sn, usp, ccp, vf, cst, drsp)

    return (
        dr.reshape(lead + (RULE,)),
        sig.reshape(lead + (SIG,)),
        mconf.reshape(lead + (1,)),
        w4,
        tw.reshape(lead + (1,)),
    )


# speedup vs baseline: 1.3035x; 1.0647x over previous
import math

import jax
import jax.numpy as jnp
from jax.experimental import pallas as pl
from jax.experimental.pallas import tpu as pltpu

U = 64
B = 128
SIG = 64
RULE = 64
CELLS = U * B
SUPPORT_MIN = 1e-4
PRIOR_MIN_POP = 4.0
PRIOR_SOFT_CAP = 0.75
USAGE_SCALE = 0.5
CONF_SCALE = 0.5
SIG_SCALE = 1.0
RET_TEMP = 1.0
SPARSE_BOOST = 1.0

T = 256


def _prologue(sigT_ref, sup_ref, conf_ref, sn_out, usp_out, ccp_out, vf_out, cst_out):
    f32 = jnp.float32
    sup = sup_ref[:]
    valid = sup > SUPPORT_MIN
    vf = valid.astype(f32)
    occ = jnp.sum(vf, keepdims=True)
    ps = jnp.clip(occ / PRIOR_MIN_POP, 0.0, 1.0)
    sp = jnp.log1p(sup)
    sp = sp / jnp.maximum(jnp.max(sp, keepdims=True), 1.0)
    sp = jnp.clip(sp * ps, 0.0, PRIOR_SOFT_CAP)
    cp = conf_ref[:] / jnp.maximum(jnp.max(conf_ref[:], keepdims=True), 1e-6)
    cp = jnp.clip(cp * ps, 0.0, PRIOR_SOFT_CAP)
    eff_temp = RET_TEMP * (1.0 + SPARSE_BOOST * (1.0 - ps))
    it = 1.0 / jnp.maximum(eff_temp, 1e-6)

    se = sigT_ref[:] + 1e-6
    nrm = jnp.maximum(jnp.sqrt(jnp.sum(se * se, axis=0, keepdims=True)), 1e-12)
    sn_out[:] = se / nrm
    usp_out[:] = USAGE_SCALE * sp
    ccp_out[:] = CONF_SCALE * cp
    vf_out[:] = vf
    lane = jax.lax.broadcasted_iota(jnp.int32, (1, 128), 1)
    cst_out[:] = ps * (lane == 0) + it * (lane == 1)


def _main(qu_ref, qb_ref, qs_ref, sn_ref, usp_ref, ccp_ref, vf_ref,
          cst_ref, drsp_ref, dr_out, sig_out, conf_out, w_out, tw_out):
    f32 = jnp.float32
    ps = cst_ref[0:1, 0:1]
    it = cst_ref[0:1, 1:2]

    qs = qs_ref[:]
    qn = jnp.maximum(jnp.sqrt(jnp.sum(qs * qs, axis=1, keepdims=True)), 1e-12)
    qsn = qs / qn
    sig_score = 0.5 * (1.0 + jnp.dot(qsn, sn_ref[:],
                                     preferred_element_type=f32))

    qu = qu_ref[:]
    qb = qb_ref[:]
    joint = jnp.concatenate([qu[:, u:u + 1] * qb for u in range(U)], axis=1)
    jl = jnp.log(jnp.maximum(joint, 1e-6))
    logits = jl + usp_ref[:] + ccp_ref[:] + SIG_SCALE * sig_score
    vf = vf_ref[:]
    z = jnp.where(vf > 0, logits, -1e9) * it

    m = jnp.max(z, axis=1, keepdims=True)
    e = jnp.exp(z - m)
    w_pre = e / jnp.sum(e, axis=1, keepdims=True)
    wv = w_pre * vf
    w = wv / jnp.maximum(jnp.sum(wv, axis=1, keepdims=True), 1e-6)

    out = jnp.dot(w, drsp_ref[:], preferred_element_type=f32)
    dr_out[:] = out[:, :RULE]
    sig_out[:] = out[:, RULE:]

    tw = jnp.max(w, axis=1, keepdims=True)
    iota = jax.lax.broadcasted_iota(jnp.int32, (1, CELLS), 1)
    ti = jnp.min(jnp.where(w == tw, iota, CELLS), axis=1, keepdims=True)
    g = (2.0 * ccp_ref[:]) * sig_score
    tcs = jnp.sum(jnp.where(iota == ti, g, 0.0), axis=1, keepdims=True)
    tw_out[:] = tw
    conf_out[:] = jnp.clip(tw * tcs * ps, 0.0, 1.0)

    w_out[...] = w.reshape(1, T, U, B)


@jax.jit
def kernel(q_u, q_b, q_sigma, delta_rule_proto, signature_proto, support_ema, ema_conf):
    f32 = jnp.float32
    lead = q_u.shape[:-1]
    n = 1
    for d in lead:
        n *= d
    qu2 = q_u.reshape(n, U)
    qb2 = q_b.reshape(n, B)
    qs2 = q_sigma.reshape(n, SIG)
    sigT = signature_proto.reshape(CELLS, SIG).T
    drsp = jnp.concatenate(
        [delta_rule_proto.reshape(CELLS, RULE),
         signature_proto.reshape(CELLS, SIG)], axis=1)
    supF = support_ema.reshape(1, CELLS)
    confF = ema_conf.reshape(1, CELLS)
    sn, usp, ccp, vf, cst = pl.pallas_call(
        _prologue,
        in_specs=[pl.BlockSpec((SIG, CELLS), lambda: (0, 0)),
                  pl.BlockSpec((1, CELLS), lambda: (0, 0)),
                  pl.BlockSpec((1, CELLS), lambda: (0, 0))],
        out_specs=[pl.BlockSpec((SIG, CELLS), lambda: (0, 0)),
                   pl.BlockSpec((1, CELLS), lambda: (0, 0)),
                   pl.BlockSpec((1, CELLS), lambda: (0, 0)),
                   pl.BlockSpec((1, CELLS), lambda: (0, 0)),
                   pl.BlockSpec((1, 128), lambda: (0, 0))],
        out_shape=[
            jax.ShapeDtypeStruct((SIG, CELLS), f32),
            jax.ShapeDtypeStruct((1, CELLS), f32),
            jax.ShapeDtypeStruct((1, CELLS), f32),
            jax.ShapeDtypeStruct((1, CELLS), f32),
            jax.ShapeDtypeStruct((1, 128), f32),
        ],
    )(sigT, supF, confF)

    grid = (n // T,)
    steps_per_lead = lead[-1] // T
    full = lambda shape: pl.BlockSpec(shape, lambda i: tuple(0 for _ in shape))
    tok = lambda width: pl.BlockSpec((T, width), lambda i: (i, 0))

    dr, sig, mconf, w4, tw = pl.pallas_call(
        _main,
        grid=grid,
        in_specs=[
            tok(U), tok(B), tok(SIG),
            full((SIG, CELLS)),
            full((1, CELLS)), full((1, CELLS)), full((1, CELLS)),
            full((1, 128)), full((CELLS, 128)),
        ],
        out_specs=[
            tok(RULE), tok(SIG), tok(1),
            pl.BlockSpec((1, T, U, B),
                         lambda i: (i // steps_per_lead, i % steps_per_lead, 0, 0)),
            tok(1),
        ],
        out_shape=[
            jax.ShapeDtypeStruct((n, RULE), f32),
            jax.ShapeDtypeStruct((n, SIG), f32),
            jax.ShapeDtypeStruct((n, 1), f32),
            jax.ShapeDtypeStruct(lead + (U, B), f32),
            jax.ShapeDtypeStruct((n, 1), f32),
        ],
        compiler_params=pltpu.CompilerParams(
            dimension_semantics=("parallel",),
            vmem_limit_bytes=110 * 1024 * 1024),
    )(qu2, qb2, qs2, sn, usp, ccp, vf, cst, drsp)

    return (
        dr.reshape(lead + (RULE,)),
        sig.reshape(lead + (SIG,)),
        mconf.reshape(lead + (1,)),
        w4,
        tw.reshape(lead + (1,)),
    )


# speedup vs baseline: 1.3672x; 1.0489x over previous
import math

import jax
import jax.numpy as jnp
from jax.experimental import pallas as pl
from jax.experimental.pallas import tpu as pltpu

U = 64
B = 128
SIG = 64
RULE = 64
CELLS = U * B
SUPPORT_MIN = 1e-4
PRIOR_MIN_POP = 4.0
PRIOR_SOFT_CAP = 0.75
USAGE_SCALE = 0.5
CONF_SCALE = 0.5
SIG_SCALE = 1.0
RET_TEMP = 1.0
SPARSE_BOOST = 1.0

T = 256


def _prologue(sigT_ref, sup_ref, conf_ref, sn_out, usp_out, ccp_out, vf_out, cst_out):
    f32 = jnp.float32
    sup = sup_ref[:]
    valid = sup > SUPPORT_MIN
    vf = valid.astype(f32)
    occ = jnp.sum(vf, keepdims=True)
    ps = jnp.clip(occ / PRIOR_MIN_POP, 0.0, 1.0)
    sp = jnp.log1p(sup)
    sp = sp / jnp.maximum(jnp.max(sp, keepdims=True), 1.0)
    sp = jnp.clip(sp * ps, 0.0, PRIOR_SOFT_CAP)
    cp = conf_ref[:] / jnp.maximum(jnp.max(conf_ref[:], keepdims=True), 1e-6)
    cp = jnp.clip(cp * ps, 0.0, PRIOR_SOFT_CAP)
    eff_temp = RET_TEMP * (1.0 + SPARSE_BOOST * (1.0 - ps))
    it = 1.0 / jnp.maximum(eff_temp, 1e-6)

    se = sigT_ref[:] + 1e-6
    nrm = jnp.maximum(jnp.sqrt(jnp.sum(se * se, axis=0, keepdims=True)), 1e-12)
    sn_out[:] = se / nrm
    usp_out[:] = USAGE_SCALE * sp
    ccp_out[:] = CONF_SCALE * cp
    vf_out[:] = vf
    lane = jax.lax.broadcasted_iota(jnp.int32, (1, 128), 1)
    cst_out[:] = ps * (lane == 0) + it * (lane == 1)


def _main(qu_ref, qb_ref, qs_ref, sn_ref, usp_ref, ccp_ref, vf_ref,
          cst_ref, drsp_ref, dr_out, sig_out, conf_out, w_out, tw_out):
    f32 = jnp.float32
    ps = cst_ref[0:1, 0:1]
    it = cst_ref[0:1, 1:2]

    qs = qs_ref[:]
    qn = jnp.maximum(jnp.sqrt(jnp.sum(qs * qs, axis=1, keepdims=True)), 1e-12)
    qsn = qs / qn
    sig_score = 0.5 * (1.0 + jnp.dot(qsn, sn_ref[:],
                                     preferred_element_type=f32))

    qu = qu_ref[:]
    qb = qb_ref[:]
    joint = jnp.concatenate([qu[:, u:u + 1] * qb for u in range(U)], axis=1)
    jl = jnp.log(jnp.maximum(joint, 1e-6))
    logits = jl + usp_ref[:] + ccp_ref[:] + SIG_SCALE * sig_score
    vf = vf_ref[:]
    z = jnp.where(vf > 0, logits, -1e9) * it

    m = jnp.max(z, axis=1, keepdims=True)
    e = jnp.exp(z - m)
    s1 = jnp.sum(e, axis=1, keepdims=True)
    ev = e * vf
    sv = jnp.sum(ev, axis=1, keepdims=True)
    w = ev * (1.0 / jnp.maximum(sv, 1e-6 * s1))

    out = jnp.dot(w, drsp_ref[:], preferred_element_type=f32)
    dr_out[:] = out[:, :RULE]
    sig_out[:] = out[:, RULE:]

    tw = jnp.max(w, axis=1, keepdims=True)
    iota = jax.lax.broadcasted_iota(jnp.int32, (1, CELLS), 1)
    ti = jnp.min(jnp.where(w == tw, iota, CELLS), axis=1, keepdims=True)
    g = (2.0 * ccp_ref[:]) * sig_score
    tcs = jnp.sum(jnp.where(iota == ti, g, 0.0), axis=1, keepdims=True)
    tw_out[:] = tw
    conf_out[:] = jnp.clip(tw * tcs * ps, 0.0, 1.0)

    w_out[...] = w.reshape(1, T, U, B)


@jax.jit
def kernel(q_u, q_b, q_sigma, delta_rule_proto, signature_proto, support_ema, ema_conf):
    f32 = jnp.float32
    lead = q_u.shape[:-1]
    n = 1
    for d in lead:
        n *= d
    qu2 = q_u.reshape(n, U)
    qb2 = q_b.reshape(n, B)
    qs2 = q_sigma.reshape(n, SIG)
    sigT = signature_proto.reshape(CELLS, SIG).T
    drsp = jnp.concatenate(
        [delta_rule_proto.reshape(CELLS, RULE),
         signature_proto.reshape(CELLS, SIG)], axis=1)
    supF = support_ema.reshape(1, CELLS)
    confF = ema_conf.reshape(1, CELLS)
    sn, usp, ccp, vf, cst = pl.pallas_call(
        _prologue,
        in_specs=[pl.BlockSpec((SIG, CELLS), lambda: (0, 0)),
                  pl.BlockSpec((1, CELLS), lambda: (0, 0)),
                  pl.BlockSpec((1, CELLS), lambda: (0, 0))],
        out_specs=[pl.BlockSpec((SIG, CELLS), lambda: (0, 0)),
                   pl.BlockSpec((1, CELLS), lambda: (0, 0)),
                   pl.BlockSpec((1, CELLS), lambda: (0, 0)),
                   pl.BlockSpec((1, CELLS), lambda: (0, 0)),
                   pl.BlockSpec((1, 128), lambda: (0, 0))],
        out_shape=[
            jax.ShapeDtypeStruct((SIG, CELLS), f32),
            jax.ShapeDtypeStruct((1, CELLS), f32),
            jax.ShapeDtypeStruct((1, CELLS), f32),
            jax.ShapeDtypeStruct((1, CELLS), f32),
            jax.ShapeDtypeStruct((1, 128), f32),
        ],
    )(sigT, supF, confF)

    grid = (n // T,)
    steps_per_lead = lead[-1] // T
    full = lambda shape: pl.BlockSpec(shape, lambda i: tuple(0 for _ in shape))
    tok = lambda width: pl.BlockSpec((T, width), lambda i: (i, 0))

    dr, sig, mconf, w4, tw = pl.pallas_call(
        _main,
        grid=grid,
        in_specs=[
            tok(U), tok(B), tok(SIG),
            full((SIG, CELLS)),
            full((1, CELLS)), full((1, CELLS)), full((1, CELLS)),
            full((1, 128)), full((CELLS, 128)),
        ],
        out_specs=[
            tok(RULE), tok(SIG), tok(1),
            pl.BlockSpec((1, T, U, B),
                         lambda i: (i // steps_per_lead, i % steps_per_lead, 0, 0)),
            tok(1),
        ],
        out_shape=[
            jax.ShapeDtypeStruct((n, RULE), f32),
            jax.ShapeDtypeStruct((n, SIG), f32),
            jax.ShapeDtypeStruct((n, 1), f32),
            jax.ShapeDtypeStruct(lead + (U, B), f32),
            jax.ShapeDtypeStruct((n, 1), f32),
        ],
        compiler_params=pltpu.CompilerParams(
            dimension_semantics=("parallel",),
            vmem_limit_bytes=110 * 1024 * 1024),
    )(qu2, qb2, qs2, sn, usp, ccp, vf, cst, drsp)

    return (
        dr.reshape(lead + (RULE,)),
        sig.reshape(lead + (SIG,)),
        mconf.reshape(lead + (1,)),
        w4,
        tw.reshape(lead + (1,)),
    )


# speedup vs baseline: 1.4522x; 1.0622x over previous
import math

import jax
import jax.numpy as jnp
from jax.experimental import pallas as pl
from jax.experimental.pallas import tpu as pltpu

U = 64
B = 128
SIG = 64
RULE = 64
CELLS = U * B
SUPPORT_MIN = 1e-4
PRIOR_MIN_POP = 4.0
PRIOR_SOFT_CAP = 0.75
USAGE_SCALE = 0.5
CONF_SCALE = 0.5
SIG_SCALE = 1.0
RET_TEMP = 1.0
SPARSE_BOOST = 1.0

T = 256


def _prologue(sigT_ref, sup_ref, conf_ref, sn_out, usp_out, ccp_out, vf_out, cst_out):
    f32 = jnp.float32
    sup = sup_ref[:]
    valid = sup > SUPPORT_MIN
    vf = valid.astype(f32)
    occ = jnp.sum(vf, keepdims=True)
    ps = jnp.clip(occ / PRIOR_MIN_POP, 0.0, 1.0)
    sp = jnp.log1p(sup)
    sp = sp / jnp.maximum(jnp.max(sp, keepdims=True), 1.0)
    sp = jnp.clip(sp * ps, 0.0, PRIOR_SOFT_CAP)
    cp = conf_ref[:] / jnp.maximum(jnp.max(conf_ref[:], keepdims=True), 1e-6)
    cp = jnp.clip(cp * ps, 0.0, PRIOR_SOFT_CAP)
    eff_temp = RET_TEMP * (1.0 + SPARSE_BOOST * (1.0 - ps))
    it = 1.0 / jnp.maximum(eff_temp, 1e-6)

    se = sigT_ref[:] + 1e-6
    nrm = jnp.maximum(jnp.sqrt(jnp.sum(se * se, axis=0, keepdims=True)), 1e-12)
    sn_out[:] = se / nrm
    usp_out[:] = USAGE_SCALE * sp
    ccp_out[:] = CONF_SCALE * cp
    vf_out[:] = vf
    lane = jax.lax.broadcasted_iota(jnp.int32, (1, 128), 1)
    cst_out[:] = ps * (lane == 0) + it * (lane == 1)


def _main(qu_ref, qb_ref, qs_ref, sn_ref, usp_ref, ccp_ref, vf_ref,
          cst_ref, drsp_ref, dr_out, sig_out, conf_out, w_out, tw_out):
    f32 = jnp.float32
    ps = cst_ref[0:1, 0:1]
    it = cst_ref[0:1, 1:2]

    qs = qs_ref[:]
    qn = jnp.maximum(jnp.sqrt(jnp.sum(qs * qs, axis=1, keepdims=True)), 1e-12)
    qsn = qs / qn
    sig_score = 0.5 * (1.0 + jnp.dot(qsn, sn_ref[:],
                                     preferred_element_type=f32))

    qu = qu_ref[:]
    qb = qb_ref[:]
    joint = jnp.concatenate([qu[:, u:u + 1] * qb for u in range(U)], axis=1)
    jl = jnp.log(jnp.maximum(joint, 1e-6))
    logits = jl + usp_ref[:] + ccp_ref[:] + SIG_SCALE * sig_score
    vf = vf_ref[:]
    z = jnp.where(vf > 0, logits, -1e9) * it

    m = jnp.max(z, axis=1, keepdims=True)
    e = jnp.exp(z - m)
    ev = e * vf
    out = jnp.dot(ev, drsp_ref[:], preferred_element_type=f32)
    rec = 1.0 / jnp.maximum(out[:, 2 * RULE:2 * RULE + 1], 1e-30)
    w = ev * rec
    dr_out[:] = out[:, :RULE] * rec
    sig_out[:] = out[:, RULE:2 * RULE] * rec

    tw = jnp.max(w, axis=1, keepdims=True)
    iota = jax.lax.broadcasted_iota(jnp.int32, (1, CELLS), 1)
    ti = jnp.min(jnp.where(w == tw, iota, CELLS), axis=1, keepdims=True)
    g = (2.0 * ccp_ref[:]) * sig_score
    tcs = jnp.sum(jnp.where(iota == ti, g, 0.0), axis=1, keepdims=True)
    tw_out[:] = tw
    conf_out[:] = jnp.clip(tw * tcs * ps, 0.0, 1.0)

    w_out[...] = w.reshape(1, T, U, B)


@jax.jit
def kernel(q_u, q_b, q_sigma, delta_rule_proto, signature_proto, support_ema, ema_conf):
    f32 = jnp.float32
    lead = q_u.shape[:-1]
    n = 1
    for d in lead:
        n *= d
    qu2 = q_u.reshape(n, U)
    qb2 = q_b.reshape(n, B)
    qs2 = q_sigma.reshape(n, SIG)
    sigT = signature_proto.reshape(CELLS, SIG).T
    drsp = jnp.concatenate(
        [delta_rule_proto.reshape(CELLS, RULE),
         signature_proto.reshape(CELLS, SIG),
         jnp.ones((CELLS, 1), f32),
         jnp.zeros((CELLS, 63), f32)], axis=1)
    supF = support_ema.reshape(1, CELLS)
    confF = ema_conf.reshape(1, CELLS)
    sn, usp, ccp, vf, cst = pl.pallas_call(
        _prologue,
        in_specs=[pl.BlockSpec((SIG, CELLS), lambda: (0, 0)),
                  pl.BlockSpec((1, CELLS), lambda: (0, 0)),
                  pl.BlockSpec((1, CELLS), lambda: (0, 0))],
        out_specs=[pl.BlockSpec((SIG, CELLS), lambda: (0, 0)),
                   pl.BlockSpec((1, CELLS), lambda: (0, 0)),
                   pl.BlockSpec((1, CELLS), lambda: (0, 0)),
                   pl.BlockSpec((1, CELLS), lambda: (0, 0)),
                   pl.BlockSpec((1, 128), lambda: (0, 0))],
        out_shape=[
            jax.ShapeDtypeStruct((SIG, CELLS), f32),
            jax.ShapeDtypeStruct((1, CELLS), f32),
            jax.ShapeDtypeStruct((1, CELLS), f32),
            jax.ShapeDtypeStruct((1, CELLS), f32),
            jax.ShapeDtypeStruct((1, 128), f32),
        ],
    )(sigT, supF, confF)

    grid = (n // T,)
    steps_per_lead = lead[-1] // T
    full = lambda shape: pl.BlockSpec(shape, lambda i: tuple(0 for _ in shape))
    tok = lambda width: pl.BlockSpec((T, width), lambda i: (i, 0))

    dr, sig, mconf, w4, tw = pl.pallas_call(
        _main,
        grid=grid,
        in_specs=[
            tok(U), tok(B), tok(SIG),
            full((SIG, CELLS)),
            full((1, CELLS)), full((1, CELLS)), full((1, CELLS)),
            full((1, 128)), full((CELLS, 192)),
        ],
        out_specs=[
            tok(RULE), tok(SIG), tok(1),
            pl.BlockSpec((1, T, U, B),
                         lambda i: (i // steps_per_lead, i % steps_per_lead, 0, 0)),
            tok(1),
        ],
        out_shape=[
            jax.ShapeDtypeStruct((n, RULE), f32),
            jax.ShapeDtypeStruct((n, SIG), f32),
            jax.ShapeDtypeStruct((n, 1), f32),
            jax.ShapeDtypeStruct(lead + (U, B), f32),
            jax.ShapeDtypeStruct((n, 1), f32),
        ],
        compiler_params=pltpu.CompilerParams(
            dimension_semantics=("parallel",),
            vmem_limit_bytes=110 * 1024 * 1024),
    )(qu2, qb2, qs2, sn, usp, ccp, vf, cst, drsp)

    return (
        dr.reshape(lead + (RULE,)),
        sig.reshape(lead + (SIG,)),
        mconf.reshape(lead + (1,)),
        w4,
        tw.reshape(lead + (1,)),
    )
